# Initial kernel scaffold; baseline (speedup 1.0000x reference)
import jax
import jax.numpy as jnp
from jax.experimental import pallas as pl


def kernel(fv, fe, fg, fv_pos, params, edge_index, node2graph, edge2graph, batch_num_nodes, batch_num_edges):
    raise NotImplementedError("write your pallas kernel here")


# speedup vs baseline: 8.3638x; 8.3638x over previous
import functools

import jax
import jax.numpy as jnp
from jax import lax
from jax.experimental import pallas as pl
from jax.experimental.pallas import tpu as pltpu
from jax.experimental.pallas import tpu_sc as plsc

F32 = jnp.float32

_N = 10000
_E = 160000
_B = 8
_PN = 1250
_PE = 20000
_DV = 256
_DE = 128
_DG = 256
_DEG = 16
_NH = 4
_DH = 64

_RN = 1000
_RE = 1000

_NC = 2
_NS = 16
_NW = _NC * _NS


_SQRT_HALF = 0.7071067690849304


def _gelu(x):
    return 0.5 * x * (1.0 + lax.erf(x * _SQRT_HALF))


def _mm(x, w):
    return lax.dot_general(x, w, (((1,), (1,)), ((), ())),
                           preferred_element_type=F32)


def _gid_select(rows, table_ref):
    gid = rows // _PN
    tab = table_ref[...]
    acc = jnp.zeros((rows.shape[0], tab.shape[1]), F32)
    for g in range(_B):
        acc = acc + jnp.where(gid == g, 1.0, 0.0) * tab[g][None, :]
    return acc


def _k1_body(fv_ref, sc8_ref, sh8_ref, lw_ref, lb_ref,
             fv1_ref, lvs_ref, lvb_ref):
    pid = pl.program_id(0)
    rows = pid * _RN + lax.broadcasted_iota(jnp.int32, (_RN, 1), 0)
    sc = _gid_select(rows, sc8_ref)
    sh = _gid_select(rows, sh8_ref)
    fv1 = fv_ref[...] * (1.0 + sc) + sh
    lv = _mm(fv1, lw_ref[...]) + lb_ref[...]
    fv1_ref[...] = fv1
    lvs_ref[...] = lv[:, :_NH * _DH]
    lvb_ref[...] = lv[:, _NH * _DH:]


def _k1(fv, sc8, sh8, lw, lb):
    grid = (_N // _RN,)
    full = lambda *s: pl.BlockSpec(s, lambda i: tuple(0 for _ in s))
    return pl.pallas_call(
        _k1_body,
        grid=grid,
        in_specs=[
            pl.BlockSpec((_RN, _DV), lambda i: (i, 0)),
            full(_B, _DV), full(_B, _DV),
            full(2 * _NH * _DH, _DV), full(1, 2 * _NH * _DH),
        ],
        out_specs=[pl.BlockSpec((_RN, _NH * _DH), lambda i: (i, 0))] * 3,
        out_shape=[jax.ShapeDtypeStruct((_N, _NH * _DH), F32)] * 3,
    )(fv, sc8, sh8, lw, lb)


def _k2_body(fvs_ref, fe_ref, lvsd_ref, lvbd_ref, ps_ref, pd_ref, gg_ref,
             g1w_ref, g1b_ref, g2w_ref, g2b_ref, g3w_ref,
             a1w_ref, a1b_ref, a2w_ref,
             m1w_ref, m1b_ref, m2w_ref, m2b_ref,
             msg_ref, feg_ref, dd_ref):
    ps = ps_ref[...]
    pd = pd_ref[...]
    d128 = ps - pd

    def _norm3(v):
        return jnp.sqrt((v[:, 0:1] * v[:, 0:1] + v[:, 1:2] * v[:, 1:2])
                        + v[:, 2:3] * v[:, 2:3])

    fe_dist = _norm3(d128)
    vs = _norm3(ps)
    vd = _norm3(pd)
    d = d128[:, :16]
    h3 = jnp.concatenate(
        [fe_dist, vs, vd, jnp.zeros((_RE, 5), F32)], axis=1)
    h = _gelu(_mm(h3, g1w_ref[...]) + g1b_ref[...])
    h = _gelu(_mm(h, g2w_ref[...]) + g2b_ref[...])
    feg = _mm(h, g3w_ref[...])
    ga = _gelu(_mm(feg, a1w_ref[...]) + a1b_ref[...])
    gate = _mm(ga, a2w_ref[...]) + gg_ref[0]
    m = jnp.concatenate([fvs_ref[...], fe_ref[...]], axis=1)
    m = _gelu(_mm(m, m1w_ref[...]) + m1b_ref[...])
    m = _gelu(_mm(m, m2w_ref[...]) + m2b_ref[...])
    msg = _gelu(lvsd_ref[...] * m + lvbd_ref[...])
    msg = jnp.concatenate(
        [msg[:, j * _DH:(j + 1) * _DH] * gate[:, j:j + 1] for j in range(_NH)],
        axis=1)
    msg_ref[...] = msg
    feg_ref[...] = feg
    lanes = lax.broadcasted_iota(jnp.int32, (_RE, 16), 1)
    dd_ref[...] = jnp.where(lanes < 3, d, jnp.where(lanes == 3, fe_dist, 0.0))


def _k2(fvs, fe, lvsd, lvbd, ps, pd, gg8, p):
    grid = (_E // _RE,)
    gpb = _PE // _RE
    full = lambda *s: pl.BlockSpec(s, lambda i: tuple(0 for _ in s))
    row = lambda d: pl.BlockSpec((_RE, d), lambda i: (i, 0))
    return pl.pallas_call(
        _k2_body,
        grid=grid,
        in_specs=[
            row(_DV), row(_DE), row(_DV), row(_DV), row(128), row(128),
            pl.BlockSpec((1, 1, _NH), lambda i: (i // gpb, 0, 0)),
            full(_DEG, 8), full(1, _DEG),
            full(_DEG, _DEG), full(1, _DEG), full(_DEG, _DEG),
            full(_DEG, _DEG), full(1, _DEG), full(_NH, _DEG),
            full(_DV, _DV + _DE), full(1, _DV),
            full(_NH * _DH, _DV), full(1, _NH * _DH),
        ],
        out_specs=[row(_DV), row(_DEG), row(16)],
        out_shape=[jax.ShapeDtypeStruct((_E, _DV), F32),
                   jax.ShapeDtypeStruct((_E, _DEG), F32),
                   jax.ShapeDtypeStruct((_E, 16), F32)],
    )(fvs, fe, lvsd, lvbd, ps, pd, gg8.reshape(_B, 1, _NH),
      jnp.pad(p['e_geo1_w'], ((0, 0), (0, 5))), p['e_geo1_b'][None, :],
      p['e_geo2_w'], p['e_geo2_b'][None, :], p['e_geo3_w'],
      p['ega1_w'], p['ega1_b'][None, :], p['ega2_w'],
      p['vmsg1_w'], p['vmsg1_b'][None, :],
      p['vmsg2_w'], p['vmsg2_b'][None, :])


def _k3_body(acc_ref, pw_ref, pb_ref, aw_ref, ab_ref, bw_ref, bb_ref,
             fv2_ref, va_ref, vb_ref):
    fv2 = _mm(acc_ref[...], pw_ref[...]) + pb_ref[...]
    fv2_ref[...] = fv2
    va_ref[...] = _mm(fv2, aw_ref[...]) + ab_ref[...]
    vb_ref[...] = _mm(fv2, bw_ref[...]) + bb_ref[...]


def _k3(nacc, p):
    grid = (_N // _RN,)
    full = lambda *s: pl.BlockSpec(s, lambda i: tuple(0 for _ in s))
    return pl.pallas_call(
        _k3_body,
        grid=grid,
        in_specs=[
            pl.BlockSpec((_RN, _NH * _DH), lambda i: (i, 0)),
            full(_DV, _NH * _DH), full(1, _DV),
            full(_DE, _DV), full(1, _DE),
            full(_DE, _DV), full(1, _DE),
        ],
        out_specs=[pl.BlockSpec((_RN, _DV), lambda i: (i, 0)),
                   pl.BlockSpec((_RN, _DE), lambda i: (i, 0)),
                   pl.BlockSpec((_RN, _DE), lambda i: (i, 0))],
        out_shape=[jax.ShapeDtypeStruct((_N, _DV), F32),
                   jax.ShapeDtypeStruct((_N, _DE), F32),
                   jax.ShapeDtypeStruct((_N, _DE), F32)],
    )(nacc, p['vproj_w'], p['vproj_b'][None, :],
      p['v2e1_w'], p['v2e1_b'][None, :], p['v2e2_w'], p['v2e2_b'][None, :])


def _k4_body(vas_ref, vbd_ref, feg_ref, fe_ref, dd_ref, fg_ref,
             u1aw_ref, u1ab_ref, u1bw_ref, u1bb_ref, u1cw_ref, u1cb_ref,
             u2aw_ref, u2ab_ref, u2bw_ref, u2bb_ref,
             p1w_ref, p1b_ref, p2w_ref, p2b_ref, p3w_ref,
             fe_out_ref, fmsg_ref):
    ab = vas_ref[...] * vbd_ref[...]
    fgrow = jnp.broadcast_to(fg_ref[0], (_RE, _DG))
    h = jnp.concatenate([ab, feg_ref[...], fgrow], axis=1)
    h = _gelu(_mm(h, u1aw_ref[...]) + u1ab_ref[...])
    h = _gelu(_mm(h, u1bw_ref[...]) + u1bb_ref[...])
    h = _mm(h, u1cw_ref[...]) + u1cb_ref[...]
    fe_shift = h[:, :_DE]
    fe_scale = h[:, _DE:]
    h2 = _mm(_gelu(_mm(fe_ref[...], u2aw_ref[...]) + u2ab_ref[...]),
             u2bw_ref[...]) + u2bb_ref[...]
    fe_new = h2 * (fe_scale + 1.0) + fe_shift
    fe_out_ref[...] = fe_new
    pm = _gelu(_mm(fe_new, p1w_ref[...]) + p1b_ref[...])
    pm = _gelu(_mm(pm, p2w_ref[...]) + p2b_ref[...])
    pm = jnp.sum(pm * p3w_ref[...], axis=1, keepdims=True)
    dd = dd_ref[...]
    fe_dist = dd[:, 3:4]
    lanes = lax.broadcasted_iota(jnp.int32, (_RE, 16), 1)
    fm16 = jnp.where(lanes < 3, pm * dd / (fe_dist + 1.0), 0.0)
    fmsg_ref[...] = jnp.concatenate(
        [fm16, jnp.zeros((_RE, 112), F32)], axis=1)


def _k4(vas, vbd, feg, fe, dd, fg, p):
    grid = (_E // _RE,)
    gpb = _PE // _RE
    full = lambda *s: pl.BlockSpec(s, lambda i: tuple(0 for _ in s))
    row = lambda d: pl.BlockSpec((_RE, d), lambda i: (i, 0))
    return pl.pallas_call(
        _k4_body,
        grid=grid,
        in_specs=[
            row(_DE), row(_DE), row(_DEG), row(_DE), row(16),
            pl.BlockSpec((1, 1, _DG), lambda i: (i // gpb, 0, 0)),
            full(_DE, _DE + _DEG + _DG), full(1, _DE),
            full(_DE, _DE), full(1, _DE),
            full(2 * _DE, _DE), full(1, 2 * _DE),
            full(_DE, _DE), full(1, _DE),
            full(_DE, _DE), full(1, _DE),
            full(_DE, _DE), full(1, _DE),
            full(_DE, _DE), full(1, _DE),
            full(1, _DE),
        ],
        out_specs=[row(_DE), row(128)],
        out_shape=[jax.ShapeDtypeStruct((_E, _DE), F32),
                   jax.ShapeDtypeStruct((_E, 128), F32)],
    )(vas, vbd, feg, fe, dd, fg.reshape(_B, 1, _DG),
      p['eu1a_w'], p['eu1a_b'][None, :], p['eu1b_w'], p['eu1b_b'][None, :],
      p['eu1c_w'], p['eu1c_b'][None, :],
      p['eu2a_w'], p['eu2a_b'][None, :], p['eu2b_w'], p['eu2b_b'][None, :],
      p['pos1_w'], p['pos1_b'][None, :], p['pos2_w'], p['pos2_b'][None, :],
      p['pos3_w'])


def _seg_body(x_ref, s_ref, mn_ref, mx_ref):
    j = pl.program_id(1)
    x = x_ref[0]
    s = jnp.sum(x, axis=0, keepdims=True)[None]
    mn = jnp.min(x, axis=0, keepdims=True)[None]
    mx = jnp.max(x, axis=0, keepdims=True)[None]

    @pl.when(j == 0)
    def _():
        s_ref[...] = s
        mn_ref[...] = mn
        mx_ref[...] = mx

    @pl.when(j > 0)
    def _():
        s_ref[...] = s_ref[...] + s
        mn_ref[...] = jnp.minimum(mn_ref[...], mn)
        mx_ref[...] = jnp.maximum(mx_ref[...], mx)


def _seg_reduce(x, seg_len, row_blk):
    d = x.shape[1]
    nb = seg_len // row_blk
    x3 = x.reshape(_B * nb, row_blk, d)
    outs = pl.pallas_call(
        _seg_body,
        grid=(_B, nb),
        in_specs=[pl.BlockSpec((1, row_blk, d), lambda g, j: (g * nb + j, 0, 0))],
        out_specs=[pl.BlockSpec((1, 1, d), lambda g, j: (g, 0, 0))] * 3,
        out_shape=[jax.ShapeDtypeStruct((_B, 1, d), F32)] * 3,
    )(x3)
    return tuple(o.reshape(_B, d) for o in outs)


def _k6_body(pos_ref, dl_ref, dm8_ref, out_ref):
    pid = pl.program_id(0)
    rows = pid * _RN + lax.broadcasted_iota(jnp.int32, (_RN, 1), 0)
    dm = _gid_select(rows, dm8_ref)
    out_ref[...] = pos_ref[...] + dl_ref[...][:, :3] - dm[:, :3]


def _k6(fv_pos, delta, dmean8):
    grid = (_N // _RN,)
    return pl.pallas_call(
        _k6_body,
        grid=grid,
        in_specs=[
            pl.BlockSpec((_RN, 3), lambda i: (i, 0)),
            pl.BlockSpec((_RN, 128), lambda i: (i, 0)),
            pl.BlockSpec((_B, 128), lambda i: (0, 0)),
        ],
        out_specs=pl.BlockSpec((_RN, 3), lambda i: (i, 0)),
        out_shape=jax.ShapeDtypeStruct((_N, 3), F32),
    )(fv_pos, delta, dmean8)


def _sc_gather(idx, tables):
    ntab = len(tables)
    e = idx.shape[0]
    per_w = e // _NW
    c = 128
    nfull = per_w // c
    tail = per_w - nfull * c
    dims = [int(t.shape[1]) for t in tables]

    scratch = [pltpu.VMEM((c,), jnp.int32), pltpu.VMEM((tail,), jnp.int32)]
    for d in dims:
        scratch.append(pltpu.VMEM((c, d), F32))
        scratch.append(pltpu.VMEM((tail, d), F32))
    scratch.append(pltpu.SemaphoreType.DMA)

    mesh = plsc.VectorSubcoreMesh(core_axis_name="c", subcore_axis_name="s")

    @functools.partial(
        pl.kernel, mesh=mesh,
        out_type=[jax.ShapeDtypeStruct((e, d), F32) for d in dims],
        scratch_types=scratch,
    )
    def k(idx_hbm, *rest):
        tabs = rest[:ntab]
        outs = rest[ntab:2 * ntab]
        sc = rest[2 * ntab:]
        idx_v, idx_t = sc[0], sc[1]
        bufs = sc[2:2 + 2 * ntab]
        sem = sc[2 + 2 * ntab]
        wid = lax.axis_index("s") * _NC + lax.axis_index("c")
        base0 = wid * per_w

        def do_chunk(start, iv, which):
            pltpu.sync_copy(idx_hbm.at[pl.ds(start, iv.shape[0])], iv)
            cps = []
            for t in range(ntab):
                buf = bufs[2 * t + which]
                cps.append(pltpu.async_copy(tabs[t].at[iv], buf, sem))
            for t in range(ntab):
                cps[t].wait()
            for t in range(ntab):
                pltpu.sync_copy(bufs[2 * t + which],
                                outs[t].at[pl.ds(start, iv.shape[0])])

        def body(i, carry):
            do_chunk(base0 + i * c, idx_v, 0)
            return carry

        lax.fori_loop(0, nfull, body, 0)
        do_chunk(base0 + nfull * c, idx_t, 1)

    res = k(idx, *tables)
    return res if isinstance(res, (list, tuple)) else (res,)


def _scat_body(x_ref, dst_ref, out_ref):
    j = pl.program_id(1)
    g = pl.program_id(0)
    dl = dst_ref[0, 0] - g * _PN
    nid = lax.broadcasted_iota(jnp.int32, (_RE, _PN), 1)
    oh = jnp.where(dl[:, None] == nid, 1.0, 0.0)
    part = lax.dot_general(oh, x_ref[...], (((0,), (0,)), ((), ())),
                           preferred_element_type=F32,
                           precision=lax.Precision.HIGHEST)

    @pl.when(j == 0)
    def _():
        out_ref[...] = part[None]

    @pl.when(j > 0)
    def _():
        out_ref[...] = out_ref[...] + part[None]


def _tc_scatter_add(x, dst):
    d = int(x.shape[1])
    gpb = _PE // _RE
    dst3 = dst.reshape(_E // _RE, 1, _RE)
    out = pl.pallas_call(
        _scat_body,
        grid=(_B, gpb),
        in_specs=[
            pl.BlockSpec((_RE, d), lambda g, j: (g * gpb + j, 0)),
            pl.BlockSpec((1, 1, _RE), lambda g, j: (g * gpb + j, 0, 0)),
        ],
        out_specs=pl.BlockSpec((1, _PN, d), lambda g, j: (g, 0, 0)),
        out_shape=jax.ShapeDtypeStruct((_B, _PN, d), F32),
    )(x, dst3)
    return out.reshape(_N, d)


def _lin(x, w, b=None):
    y = x @ w.T
    if b is not None:
        y = y + b
    return y


def kernel(fv, fe, fg, fv_pos, params, edge_index, node2graph, edge2graph,
           batch_num_nodes, batch_num_edges):
    p = params
    src = edge_index[0].astype(jnp.int32)
    dst = edge_index[1].astype(jnp.int32)

    sc_sh = _lin(fg, p['g2v_w'], p['g2v_b'])
    sc8, sh8 = sc_sh[:, :_DV], sc_sh[:, _DV:]
    gg8 = _lin(jax.nn.gelu(_lin(fg, p['ga1_w'], p['ga1_b']),
                           approximate=False), p['ga2_w'])
    pos_pad = jnp.pad(fv_pos, ((0, 0), (0, 125)))

    fv1, lvs, lvb = _k1(fv, sc8, sh8, p['lvu_w'], p['lvu_b'][None, :])

    fvs, ps = _sc_gather(src, [fv1, pos_pad])
    lvsd, lvbd, pd = _sc_gather(dst, [lvs, lvb, pos_pad])

    msg, feg, dd = _k2(fvs, fe, lvsd, lvbd, ps, pd, gg8, p)

    nacc = _tc_scatter_add(msg, dst)

    fv2, va, vb = _k3(nacc, p)

    (vas,) = _sc_gather(src, [va])
    (vbd,) = _sc_gather(dst, [vb])

    fe_new, fmsg = _k4(vas, vbd, feg, fe, dd, fg, p)

    delta = _tc_scatter_add(fmsg, dst)

    se, mne, mxe = _seg_reduce(fe_new, _PE, 2000)
    sv, mnv, mxv = _seg_reduce(fv2, _PN, 1250)
    sg, mng, mxg = _seg_reduce(feg, _PE, 2000)
    sd, _, _ = _seg_reduce(delta, _PN, 1250)

    cnt_e = batch_num_edges.astype(F32)
    cnt_n = batch_num_nodes.astype(F32)
    fe2g = (_lin(se / cnt_e[:, None], p['er1_w'], p['er1_b'])
            + _lin(mne, p['er2_w'], p['er2_b'])
            + _lin(mxe, p['er3_w'], p['er3_b']))
    fv2g = (_lin(sv / cnt_n[:, None], p['vr1_w'], p['vr1_b'])
            + _lin(mnv, p['vr2_w'], p['vr2_b'])
            + _lin(mxv, p['vr3_w'], p['vr3_b']))
    feg2g = (_lin(sg / cnt_e[:, None], p['egr1_w'], p['egr1_b'])
             + _lin(mng, p['egr2_w'], p['egr2_b'])
             + _lin(mxg, p['egr3_w'], p['egr3_b']))
    gcat = jnp.concatenate([fe2g, fv2g, feg2g, cnt_n[:, None]], axis=1)
    gx = _lin(jax.nn.gelu(_lin(gcat, p['gu1_w'], p['gu1_b']),
                          approximate=False), p['gu2_w'], p['gu2_b'])
    gi = _lin(gx, p['gru_ih_w'], p['gru_ih_b'])
    gh = _lin(fg, p['gru_hh_w'], p['gru_hh_b'])
    i_r, i_z, i_n = jnp.split(gi, 3, axis=1)
    h_r, h_z, h_n = jnp.split(gh, 3, axis=1)
    r = jax.nn.sigmoid(i_r + h_r)
    z = jax.nn.sigmoid(i_z + h_z)
    n = jnp.tanh(i_n + r * h_n)
    fg_new = (1.0 - z) * n + z * fg

    dmean8 = sd / cnt_n[:, None]
    fv_pos_new = _k6(fv_pos, delta, dmean8)

    return (fv2, fe_new, fg_new, fv_pos_new)
